# baseline (speedup 1.0000x reference)
import functools

import jax
import jax.numpy as jnp
from jax import lax
from jax.experimental import pallas as pl
from jax.experimental.pallas import tpu as pltpu
from jax.experimental.pallas import tpu_sc as plsc

_NC = 2
_NS = 16
_NW = _NC * _NS
_CHUNK = 128
_NBUF = 2


@functools.lru_cache(maxsize=None)
def _build(b, vocab, dim):
    n_chunks = -(-b // _CHUNK)
    rem = b - (b // _CHUNK) * _CHUNK
    k_max = -(-n_chunks // _NW)
    mesh = plsc.VectorSubcoreMesh(core_axis_name="c", subcore_axis_name="s")

    @functools.partial(
        pl.kernel,
        mesh=mesh,
        compiler_params=pltpu.CompilerParams(
            use_tc_tiling_on_sc=False, needs_layout_passes=False
        ),
        out_type=jax.ShapeDtypeStruct((dim, b), jnp.float32),
        scratch_types=[
            pltpu.VMEM((_NBUF, _CHUNK), jnp.int32),
            pltpu.VMEM((_NBUF, _CHUNK, dim), jnp.float32),
            pltpu.VMEM((dim, _CHUNK), jnp.float32),
            pltpu.SemaphoreType.DMA,
            pltpu.SemaphoreType.DMA,
        ],
    )
    def _gather(idx_hbm, table_hbm, out_hbm, idx_v, rows_v, tr_v, gsem0, gsem1):
        wid = lax.axis_index("s") * _NC + lax.axis_index("c")
        gsems = (gsem0, gsem1)

        def start_chunk(c, slot):
            pltpu.sync_copy(idx_hbm.at[c], idx_v.at[slot])
            pltpu.async_copy(
                table_hbm.at[idx_v.at[slot]], rows_v.at[slot], gsems[slot]
            )

        def finish_chunk(c, slot):
            pltpu.make_async_copy(
                table_hbm.at[idx_v.at[slot]], rows_v.at[slot], gsems[slot]
            ).wait()
            for j in range(dim):
                cidx = jnp.full((16,), j, jnp.int32)
                for kk in range(_CHUNK // 16):
                    ridx = kk * 16 + lax.iota(jnp.int32, 16)
                    tr_v[j, pl.ds(kk * 16, 16)] = plsc.load_gather(
                        rows_v.at[slot], [ridx, cidx]
                    )
            base = c * _CHUNK
            if rem:
                @pl.when(c == n_chunks - 1)
                def _():
                    pltpu.sync_copy(
                        tr_v.at[:, pl.ds(0, rem)],
                        out_hbm.at[:, pl.ds(base, rem)],
                    )

                @pl.when(c != n_chunks - 1)
                def _():
                    pltpu.sync_copy(tr_v, out_hbm.at[:, pl.ds(base, _CHUNK)])
            else:
                pltpu.sync_copy(tr_v, out_hbm.at[:, pl.ds(base, _CHUNK)])

        @pl.when(wid < n_chunks)
        def _():
            start_chunk(wid, 0)

        @pl.loop(0, k_max, step=_NBUF)
        def _chunks(k):
            for b_ in range(_NBUF):
                c = wid + (k + b_) * _NW
                nxt = c + _NW
                slot = b_ % _NBUF

                @pl.when(nxt < n_chunks)
                def _(nxt=nxt, slot=slot):
                    start_chunk(nxt, (slot + 1) % _NBUF)

                @pl.when(c < n_chunks)
                def _(c=c, slot=slot):
                    finish_chunk(c, slot)

    return _gather


def kernel(token_id, table):
    b = token_id.shape[0]
    vocab, dim = table.shape
    n_chunks = -(-b // _CHUNK)
    b_pad = n_chunks * _CHUNK
    idx = token_id.astype(jnp.int32)
    npad = b_pad - b
    if npad:
        pad = jnp.arange(npad, dtype=jnp.int32) % jnp.int32(vocab)
        idx = jnp.concatenate([idx, pad])
    idx2 = idx.reshape(n_chunks, _CHUNK)
    out_t = _build(b, vocab, dim)(idx2, table)
    return out_t.T
